# baseline (speedup 1.0000x reference)
import functools

import jax
import jax.numpy as jnp
from jax import lax
from jax.experimental import pallas as pl
from jax.experimental.pallas import tpu as pltpu
from jax.experimental.pallas import tpu_sc as plsc

DIM = 64
NC = 2
NS = 16
NW = NC * NS
LANES = 16
CHUNK = 128
NBUF = 5


@functools.lru_cache(maxsize=None)
def _emb_kernel(N):
    PER_W = N // NW
    NCHUNK = PER_W // CHUNK
    NOUTER = NCHUNK // NBUF - 1
    mesh = plsc.VectorSubcoreMesh(core_axis_name="c", subcore_axis_name="s")
    out_t = jax.ShapeDtypeStruct((N, DIM), jnp.float32)
    scratch = (
        [pltpu.VMEM((PER_W,), jnp.int32), pltpu.VMEM((2, LANES), jnp.float32)]
        + [pltpu.VMEM((CHUNK, DIM), jnp.float32) for _ in range(NBUF)]
        + [pltpu.SemaphoreType.DMA for _ in range(NBUF)]
    )

    @functools.partial(
        pl.kernel,
        mesh=mesh,
        out_type=(out_t, out_t),
        scratch_types=scratch,
        compiler_params=pltpu.CompilerParams(use_tc_tiling_on_sc=False),
    )
    def k(enc_idx, dec_idx, scales, table, enc_out, dec_out, idx_v, scale_v, *rest):
        bufs = rest[:NBUF]
        sems = rest[NBUF:]
        wid = lax.axis_index("s") * NC + lax.axis_index("c")
        base = wid * PER_W
        pltpu.sync_copy(scales, scale_v)

        def issue(b, c):
            pltpu.async_copy(
                table.at[idx_v.at[pl.ds(c * CHUNK, CHUNK)]], bufs[b], sems[b]
            )

        def wait(b):
            pltpu.make_async_copy(table.at[pl.ds(0, CHUNK)], bufs[b], sems[b]).wait()

        for side, (idx_h, out_h) in enumerate(((enc_idx, enc_out), (dec_idx, dec_out))):
            s = scale_v[side]
            pltpu.sync_copy(idx_h.at[pl.ds(base, PER_W)], idx_v)
            for b in range(NBUF):
                issue(b, b)

            def process(b, c):
                wait(b)

                def mul_row(i, _):
                    for j in range(DIM // LANES):
                        sl = pl.ds(j * LANES, LANES)
                        bufs[b][i, sl] = bufs[b][i, sl] * s
                    return 0

                lax.fori_loop(0, CHUNK, mul_row, 0)
                pltpu.sync_copy(bufs[b], out_h.at[pl.ds(base + c * CHUNK, CHUNK)])

            def outer(o, _):
                for b in range(NBUF):
                    c = o * NBUF + b
                    process(b, c)
                    issue(b, c + NBUF)
                return 0

            lax.fori_loop(0, NOUTER, outer, 0)
            for b in range(NBUF):
                process(b, NOUTER * NBUF + b)

    return k


def kernel(input_ids, encoder_embed_scale, decoder_input_ids, decoder_embed_scale, table):
    b, l = input_ids.shape
    n = b * l
    enc_idx = input_ids.reshape(n).astype(jnp.int32)
    dec_idx = decoder_input_ids.reshape(n).astype(jnp.int32)
    scales = jnp.stack(
        [
            jnp.full((LANES,), encoder_embed_scale, jnp.float32),
            jnp.full((LANES,), decoder_embed_scale, jnp.float32),
        ]
    )
    enc_out, dec_out = _emb_kernel(n)(enc_idx, dec_idx, scales, table)
    return (enc_out.reshape(b, l, DIM), dec_out.reshape(b, l, DIM))
